# baseline (speedup 1.0000x reference)
import functools

import jax
import jax.numpy as jnp
from jax import lax
from jax.experimental import pallas as pl
from jax.experimental.pallas import tpu as pltpu
from jax.experimental.pallas import tpu_sc as plsc

_LANES = 128
_ROWS_PER_CHUNK = 8
_CHUNK = _LANES * _ROWS_PER_CHUNK
_DPAD = 112


@functools.lru_cache(maxsize=None)
def _build_gather(V, D, B):
    info = plsc.get_sparse_core_info()
    NC, NS = info.num_cores, info.num_subcores
    NW = NC * NS
    n_rows = B // _LANES
    rows_per_w = n_rows // NW
    n_chunks = rows_per_w // _ROWS_PER_CHUNK
    mesh = plsc.VectorSubcoreMesh(core_axis_name="c", subcore_axis_name="s")

    @functools.partial(
        pl.kernel,
        mesh=mesh,
        out_type=jax.ShapeDtypeStruct((B, D), jnp.float32),
        scratch_types=[
            pltpu.VMEM((_ROWS_PER_CHUNK, _LANES), jnp.int32),
            pltpu.VMEM((_CHUNK, D), jnp.float32),
            pltpu.SemaphoreType.DMA,
        ],
        compiler_params=pltpu.CompilerParams(use_tc_tiling_on_sc=False),
    )
    def gather_kernel(table_hbm, idx_hbm, out_hbm, idx_v, rows_v, sem):
        wid = lax.axis_index("s") * NC + lax.axis_index("c")
        row_base = wid * rows_per_w

        def chunk_body(ci, carry):
            row_off = row_base + ci * _ROWS_PER_CHUNK
            pltpu.sync_copy(idx_hbm.at[pl.ds(row_off, _ROWS_PER_CHUNK)], idx_v)
            copies = [
                pltpu.async_copy(
                    table_hbm.at[idx_v.at[j]],
                    rows_v.at[pl.ds(j * _LANES, _LANES)],
                    sem,
                )
                for j in range(_ROWS_PER_CHUNK)
            ]
            for c in copies:
                c.wait()
            pltpu.sync_copy(rows_v, out_hbm.at[pl.ds(row_off * _LANES, _CHUNK)])
            return carry

        lax.fori_loop(0, n_chunks, chunk_body, 0)

    return gather_kernel


def kernel(sen, word_embeddings):
    Bx, L = sen.shape
    V, D = word_embeddings.shape
    B = Bx * L
    idx = sen.reshape(B // _LANES, _LANES)
    table = jnp.pad(word_embeddings, ((0, 0), (0, _DPAD - D)))
    out = _build_gather(V, _DPAD, B)(table, idx)
    return out[:, :D].reshape(Bx, L, D, 1)
